# Initial kernel scaffold; baseline (speedup 1.0000x reference)
import jax
import jax.numpy as jnp
from jax.experimental import pallas as pl


def kernel(x, edge_index, W_z, b_z, Wl_z, bl_z, W_r, b_r, Wl_r, bl_r, W_h, b_h, Wl_h, bl_h, att, W_out, b_out):
    raise NotImplementedError("write your pallas kernel here")


# speedup vs baseline: 187.7017x; 187.7017x over previous
import functools

import jax
import jax.numpy as jnp
from jax import lax
from jax.experimental import pallas as pl
from jax.experimental.pallas import tpu as pltpu
from jax.experimental.pallas import tpu_sc as plsc

N = 10000
E = 320000
F_IN = 2
HID = 32
P = 12

NC = 2
NS = 16
NW = NC * NS
C = 128
CHUNKS = 79
E_PT = C * CHUNKS
E_PAD = E_PT * NW
D = 32
N_PAD = 10112
SLAB = N_PAD // NS

_mesh = plsc.VectorSubcoreMesh(core_axis_name="c", subcore_axis_name="s")
_sc_params = pltpu.CompilerParams(use_tc_tiling_on_sc=False)


@functools.partial(
    pl.kernel,
    mesh=_mesh,
    compiler_params=_sc_params,
    out_type=jax.ShapeDtypeStruct((NC, N_PAD, 16), jnp.float32),
    scratch_types=[
        pltpu.VMEM((C,), jnp.int32),
        pltpu.VMEM((C, 16), jnp.float32),
        pltpu.VMEM((SLAB, 16), jnp.float32),
        pltpu.VMEM_SHARED((N_PAD, 16), jnp.float32),
    ],
)
def _hist(dst_hbm, out_hbm, idx_v, ones_v, slab_v, deg_sh):
    cid = lax.axis_index("c")
    sid = lax.axis_index("s")
    wid = cid * NS + sid

    def fill_ones(i, carry):
        ones_v[i, :] = jnp.full((16,), 1.0, jnp.float32)
        return carry

    lax.fori_loop(0, C, fill_ones, 0)

    def fill_zero(i, carry):
        slab_v[i, :] = jnp.zeros((16,), jnp.float32)
        return carry

    lax.fori_loop(0, SLAB, fill_zero, 0)
    pltpu.sync_copy(slab_v, deg_sh.at[pl.ds(sid * SLAB, SLAB)])
    plsc.subcore_barrier()

    base = wid * E_PT

    def body(j, carry):
        pltpu.sync_copy(dst_hbm.at[pl.ds(base + j * C, C)], idx_v)
        pltpu.sync_copy(ones_v, deg_sh.at[idx_v], add=True)
        return carry

    lax.fori_loop(0, CHUNKS, body, 0)
    plsc.subcore_barrier()

    pltpu.sync_copy(deg_sh.at[pl.ds(sid * SLAB, SLAB)], slab_v)
    pltpu.sync_copy(slab_v, out_hbm.at[cid, pl.ds(sid * SLAB, SLAB)])


@functools.partial(
    pl.kernel,
    mesh=_mesh,
    compiler_params=_sc_params,
    out_type=jax.ShapeDtypeStruct((NC, N_PAD, D), jnp.float32),
    scratch_types=[
        pltpu.VMEM((C,), jnp.int32),
        pltpu.VMEM((C,), jnp.int32),
        pltpu.VMEM((C, D), jnp.float32),
        pltpu.VMEM((SLAB, D), jnp.float32),
        pltpu.VMEM_SHARED((N_PAD, D), jnp.float32),
        pltpu.SemaphoreType.DMA,
    ],
)
def _agg(src_hbm, dst_hbm, xp_hbm, out_hbm, sidx_v, didx_v, rows_v, slab_v,
         y_sh, sem):
    cid = lax.axis_index("c")
    sid = lax.axis_index("s")
    wid = cid * NS + sid

    def fill_zero(i, carry):
        slab_v[i, :16] = jnp.zeros((16,), jnp.float32)
        slab_v[i, 16:32] = jnp.zeros((16,), jnp.float32)
        return carry

    lax.fori_loop(0, SLAB, fill_zero, 0)
    pltpu.sync_copy(slab_v, y_sh.at[pl.ds(sid * SLAB, SLAB)])
    plsc.subcore_barrier()

    base = wid * E_PT

    def body(j, carry):
        b = base + j * C
        pltpu.sync_copy(src_hbm.at[pl.ds(b, C)], sidx_v)
        pltpu.sync_copy(dst_hbm.at[pl.ds(b, C)], didx_v)
        pltpu.async_copy(xp_hbm.at[sidx_v], rows_v, sem).wait()
        pltpu.sync_copy(rows_v, y_sh.at[didx_v], add=True)
        return carry

    lax.fori_loop(0, CHUNKS, body, 0)
    plsc.subcore_barrier()

    pltpu.sync_copy(y_sh.at[pl.ds(sid * SLAB, SLAB)], slab_v)
    pltpu.sync_copy(slab_v, out_hbm.at[cid, pl.ds(sid * SLAB, SLAB)])


def _prep_body(x24_ref, deg_ref, xp_ref):
    degc = deg_ref[0, :N, 0:1] + deg_ref[1, :N, 0:1] + 1.0
    dinv = lax.rsqrt(degc)
    xp_ref[:, 0:F_IN * P] = x24_ref[:, :] * dinv
    xp_ref[:, F_IN * P:F_IN * P + 1] = dinv
    xp_ref[:, F_IN * P + 1:D] = jnp.zeros((N, D - F_IN * P - 1), jnp.float32)


BN = 1000


def _final_body(y2_ref, xp_ref, cz_ref, czb_ref, ch_ref, chb_ref, probs_ref,
                wout_ref, bout_ref, out_ref):
    xp = xp_ref[:, :]
    dinv = xp[:, F_IN * P:F_IN * P + 1]
    scat = y2_ref[0, :, 0:F_IN * P] + y2_ref[1, :, 0:F_IN * P]
    yg = dinv * (scat + xp[:, 0:F_IN * P])
    acc = jnp.zeros((BN, HID), jnp.float32)
    for p in range(P):
        y0 = yg[:, p:p + 1]
        y1 = yg[:, P + p:P + p + 1]
        z = jax.nn.sigmoid(y0 * cz_ref[0:1, :] + y1 * cz_ref[1:2, :]
                           + czb_ref[0:1, :])
        ht = jnp.tanh(y0 * ch_ref[0:1, :] + y1 * ch_ref[1:2, :]
                      + chb_ref[0:1, :])
        acc = acc + probs_ref[0, p] * (1.0 - z) * ht
    h = jnp.maximum(acc, 0.0)
    out_ref[:, :] = lax.dot_general(
        h, wout_ref[:, :], (((1,), (0,)), ((), ())),
        preferred_element_type=jnp.float32) + bout_ref[0:1, :]


def kernel(x, edge_index, W_z, b_z, Wl_z, bl_z, W_r, b_r, Wl_r, bl_r,
           W_h, b_h, Wl_h, bl_h, att, W_out, b_out):
    src = edge_index[0]
    dst = edge_index[1]
    pad = E_PAD - E
    src_p = jnp.concatenate([src, jnp.zeros((pad,), jnp.int32)])
    dst_p = jnp.concatenate([dst, jnp.full((pad,), N, jnp.int32)])

    deg2 = _hist(dst_p)

    x24 = x.reshape(N, F_IN * P)
    xp = pl.pallas_call(
        _prep_body,
        out_shape=jax.ShapeDtypeStruct((N, D), jnp.float32),
    )(x24, deg2)

    y2 = _agg(src_p, dst_p, xp)

    Cz = W_z @ Wl_z[:HID]
    czb = (b_z @ Wl_z[:HID] + bl_z).reshape(1, HID)
    Ch = W_h @ Wl_h[:HID]
    chb = (b_h @ Wl_h[:HID] + bl_h).reshape(1, HID)
    probs = jax.nn.softmax(att).reshape(1, P)

    full = lambda s: pl.BlockSpec(s, lambda i: (0,) * len(s))
    out = pl.pallas_call(
        _final_body,
        grid=(N // BN,),
        out_shape=jax.ShapeDtypeStruct((N, P), jnp.float32),
        in_specs=[
            pl.BlockSpec((NC, BN, D), lambda i: (0, i, 0)),
            pl.BlockSpec((BN, D), lambda i: (i, 0)),
            full((F_IN, HID)),
            full((1, HID)),
            full((F_IN, HID)),
            full((1, HID)),
            pl.BlockSpec(memory_space=pltpu.SMEM),
            full((HID, P)),
            full((1, P)),
        ],
        out_specs=pl.BlockSpec((BN, P), lambda i: (i, 0)),
    )(y2, xp, Cz, czb, Ch, chb, probs, W_out, b_out.reshape(1, P))
    return out
